# baseline (speedup 1.0000x reference)
import functools

import jax
import jax.numpy as jnp
from jax import lax
from jax.experimental import pallas as pl
from jax.experimental.pallas import tpu as pltpu
from jax.experimental.pallas import tpu_sc as plsc

N = 10000
E = 320000
D = 128
H = 128

NP = 10240
PAD = N
NC = 2
NS = 16
NW = NC * NS
EB = 128
KB = 80
EP = NW * KB * EB
RPT = NP // NS

_mesh = plsc.VectorSubcoreMesh(
    core_axis_name="c", subcore_axis_name="s", num_cores=NC, num_subcores=NS
)


@functools.partial(
    pl.kernel,
    out_type=[
        jax.ShapeDtypeStruct((NC, NP, H), jnp.float32),
        jax.ShapeDtypeStruct((NC, NP, H), jnp.float32),
    ],
    mesh=_mesh,
    scratch_types=[
        pltpu.VMEM((KB, EB), jnp.int32),
        pltpu.VMEM((KB, EB), jnp.int32),
        pltpu.VMEM((EB, H), jnp.float32),
        pltpu.VMEM_SHARED((NP, H), jnp.float32),
        pltpu.SemaphoreType.DMA,
    ],
)
def _sc_degrees(src_hbm, dst_hbm, ones_hbm, zeros_hbm, out_o, out_i,
                sidx, didx, ones, acc, sem):
    c = lax.axis_index("c")
    s = lax.axis_index("s")
    w = c * NS + s
    pltpu.sync_copy(ones_hbm, ones)
    pltpu.sync_copy(src_hbm.at[pl.ds(w * KB, KB)], sidx)
    pltpu.sync_copy(dst_hbm.at[pl.ds(w * KB, KB)], didx)

    for idx, out in ((sidx, out_o), (didx, out_i)):
        pltpu.sync_copy(zeros_hbm.at[pl.ds(s * RPT, RPT)],
                        acc.at[pl.ds(s * RPT, RPT)])
        plsc.subcore_barrier()

        def body(j, carry, idx=idx):
            pltpu.sync_copy(ones, acc.at[idx.at[j]], add=True)
            return carry

        lax.fori_loop(0, KB, body, 0)
        plsc.subcore_barrier()
        pltpu.sync_copy(acc.at[pl.ds(s * RPT, RPT)],
                        out.at[c].at[pl.ds(s * RPT, RPT)])
        plsc.subcore_barrier()


@functools.partial(
    pl.kernel,
    out_type=jax.ShapeDtypeStruct((NC, NP, H), jnp.float32),
    mesh=_mesh,
    scratch_types=[
        pltpu.VMEM((KB, EB), jnp.int32),
        pltpu.VMEM((4, EB), jnp.int32),
        pltpu.VMEM((EB, H), jnp.float32),
        pltpu.VMEM((EB, H), jnp.float32),
        pltpu.VMEM_SHARED((NP, H), jnp.float32),
        pltpu.SemaphoreType.DMA,
        pltpu.SemaphoreType.DMA,
        pltpu.SemaphoreType.DMA,
        pltpu.SemaphoreType.DMA,
    ],
)
def _sc_agg(t_hbm, src_hbm, dst_hbm, zeros_hbm, out_hbm,
            sidx, didx, rows0, rows1, acc, semr, semi, sems0, sems1):
    c = lax.axis_index("c")
    s = lax.axis_index("s")
    w = c * NS + s
    pltpu.sync_copy(zeros_hbm.at[pl.ds(s * RPT, RPT)], acc.at[pl.ds(s * RPT, RPT)])
    pltpu.sync_copy(src_hbm.at[pl.ds(w * KB, KB)], sidx)
    plsc.subcore_barrier()

    pltpu.async_copy(t_hbm.at[sidx.at[0]], rows0, semr)
    pltpu.async_copy(dst_hbm.at[w * KB], didx.at[pl.ds(0, 1)], semi)

    def body(i, carry):
        for b in range(2):
            j = 2 * i + b
            cur, nxt = (rows0, rows1) if b == 0 else (rows1, rows0)
            scur, snxt = (sems0, sems1) if b == 0 else (sems1, sems0)
            jm = lax.rem(j, 4)
            pltpu.make_async_copy(t_hbm.at[sidx.at[j]], cur, semr).wait()
            pltpu.make_async_copy(dst_hbm.at[w * KB + j],
                                  didx.at[pl.ds(jm, 1)], semi).wait()
            pltpu.async_copy(cur, acc.at[didx.at[jm]], scur, add=True)

            @pl.when(j >= 1)
            def _():
                pltpu.make_async_copy(nxt, acc.at[didx.at[0]], snxt).wait()

            @pl.when(j + 1 < KB)
            def _():
                jn = lax.rem(j + 1, 4)
                pltpu.async_copy(t_hbm.at[sidx.at[j + 1]], nxt, semr)
                pltpu.async_copy(dst_hbm.at[w * KB + j + 1],
                                 didx.at[pl.ds(jn, 1)], semi)

        return carry

    lax.fori_loop(0, KB // 2, body, 0)
    pltpu.make_async_copy(rows1, acc.at[didx.at[0]], sems1).wait()
    plsc.subcore_barrier()
    pltpu.sync_copy(acc.at[pl.ds(s * RPT, RPT)],
                    out_hbm.at[c].at[pl.ds(s * RPT, RPT)])


R = 1024


def _tc1_body(degop_ref, degip_ref, x_ref, w1_ref, nrm_ref, t1_ref):
    dego = degop_ref[0, :, 0] + degop_ref[1, :, 0]
    degi = degip_ref[0, :, 0] + degip_ref[1, :, 0]
    deg = jnp.stack([dego, degi])
    nrm = jnp.where(deg > 0, lax.rsqrt(jnp.maximum(deg, 1e-12)), 0.0)
    nrm_ref[...] = nrm
    ns_col = nrm[0, :][:, None]
    t1_ref[...] = (
        jnp.dot(x_ref[...], w1_ref[...], preferred_element_type=jnp.float32) * ns_col
    )


_tc1 = pl.pallas_call(
    _tc1_body,
    grid=(NP // R,),
    in_specs=[
        pl.BlockSpec((NC, R, H), lambda i: (0, i, 0)),
        pl.BlockSpec((NC, R, H), lambda i: (0, i, 0)),
        pl.BlockSpec((R, D), lambda i: (i, 0)),
        pl.BlockSpec((D, H), lambda i: (0, 0)),
    ],
    out_specs=[
        pl.BlockSpec((2, R), lambda i: (0, i)),
        pl.BlockSpec((R, H), lambda i: (i, 0)),
    ],
    out_shape=[
        jax.ShapeDtypeStruct((2, NP), jnp.float32),
        jax.ShapeDtypeStruct((NP, H), jnp.float32),
    ],
)


def _tc_mid_body(aggp_ref, nrm_ref, b_ref, w_ref, t_ref):
    nd_col = nrm_ref[1, :][:, None]
    h = jnp.maximum((aggp_ref[0] + aggp_ref[1]) * nd_col + b_ref[...], 0.0)
    ns_col = nrm_ref[0, :][:, None]
    t_ref[...] = (
        jnp.dot(h, w_ref[...], preferred_element_type=jnp.float32) * ns_col
    )


_tc_mid = pl.pallas_call(
    _tc_mid_body,
    grid=(NP // R,),
    in_specs=[
        pl.BlockSpec((NC, R, H), lambda i: (0, i, 0)),
        pl.BlockSpec((2, R), lambda i: (0, i)),
        pl.BlockSpec((1, H), lambda i: (0, 0)),
        pl.BlockSpec((H, H), lambda i: (0, 0)),
    ],
    out_specs=pl.BlockSpec((R, H), lambda i: (i, 0)),
    out_shape=jax.ShapeDtypeStruct((NP, H), jnp.float32),
)


def _tc3_body(aggp_ref, nrm_ref, b_ref, w3_ref, t3_ref):
    nd_col = nrm_ref[1, :][:, None]
    h = jnp.maximum((aggp_ref[0] + aggp_ref[1]) * nd_col + b_ref[...], 0.0)
    t3 = jnp.sum(h * w3_ref[...], axis=1) * nrm_ref[0, :]
    t3_ref[...] = jnp.broadcast_to(t3[:, None], (R, H))


_tc3 = pl.pallas_call(
    _tc3_body,
    grid=(NP // R,),
    in_specs=[
        pl.BlockSpec((NC, R, H), lambda i: (0, i, 0)),
        pl.BlockSpec((2, R), lambda i: (0, i)),
        pl.BlockSpec((1, H), lambda i: (0, 0)),
        pl.BlockSpec((1, H), lambda i: (0, 0)),
    ],
    out_specs=pl.BlockSpec((R, H), lambda i: (i, 0)),
    out_shape=jax.ShapeDtypeStruct((NP, H), jnp.float32),
)


def _tc4_body(aggs_ref, nrm_ref, b3_ref, y_ref):
    a = aggs_ref[0, :, 0] + aggs_ref[1, :, 0]
    v = a * nrm_ref[1, :] + b3_ref[0, 0]
    y_ref[...] = jnp.maximum(v, 0.0)[:, None]


_tc4 = pl.pallas_call(
    _tc4_body,
    grid=(NP // R,),
    in_specs=[
        pl.BlockSpec((NC, R, H), lambda i: (0, i, 0)),
        pl.BlockSpec((2, R), lambda i: (0, i)),
        pl.BlockSpec((1, 1), lambda i: (0, 0)),
    ],
    out_specs=pl.BlockSpec((R, 1), lambda i: (i, 0)),
    out_shape=jax.ShapeDtypeStruct((NP, 1), jnp.float32),
)


def kernel(features, edge_index, W1, b1, W2, b2, W3, b3):
    x = jnp.zeros((NP, D), jnp.float32).at[:N].set(features)
    padv = PAD + jnp.arange(EP - E, dtype=jnp.int32) % (NP - N)
    srcp = jnp.concatenate([edge_index[0], padv]).reshape(EP // EB, EB)
    dstp = jnp.concatenate([edge_index[1], padv]).reshape(EP // EB, EB)
    onesH = jnp.ones((EB, H), jnp.float32)
    zH = jnp.zeros((NP, H), jnp.float32)

    dst3 = dstp.reshape(EP // EB, 1, EB)

    degop, degip = _sc_degrees(srcp, dstp, onesH, zH)
    nrm, t1 = _tc1(degop, degip, x, W1)
    agg1 = _sc_agg(t1, srcp, dst3, zH)
    t2 = _tc_mid(agg1, nrm, b1.reshape(1, H), W2)
    agg2 = _sc_agg(t2, srcp, dst3, zH)
    t3 = _tc3(agg2, nrm, b2.reshape(1, H), W3.reshape(1, H))
    agg3 = _sc_agg(t3, srcp, dst3, zH)
    y = _tc4(agg3, nrm, b3.reshape(1, 1))
    return y[:N]


# speedup vs baseline: 1.0301x; 1.0301x over previous
import functools

import jax
import jax.numpy as jnp
from jax import lax
from jax.experimental import pallas as pl
from jax.experimental.pallas import tpu as pltpu
from jax.experimental.pallas import tpu_sc as plsc

N = 10000
E = 320000
D = 128
H = 128

NP = 10240
PAD = N
NC = 2
NS = 16
NW = NC * NS
EB = 128
KB = 80
EP = NW * KB * EB
RPT = NP // NS

_mesh = plsc.VectorSubcoreMesh(
    core_axis_name="c", subcore_axis_name="s", num_cores=NC, num_subcores=NS
)


def _rsqrt16(x):
    s = jnp.full((16,), 1.0, jnp.float32)
    for k in range(1, 14):
        s = jnp.where(x >= jnp.float32(2.0 ** k),
                      jnp.full((16,), 2.0 ** ((k + 1) // 2), jnp.float32), s)
    for _ in range(5):
        s = 0.5 * (s + x / s)
    y = 1.0 / s
    for _ in range(2):
        y = y * (1.5 - 0.5 * x * y * y)
    return jnp.where(x > 0, y, jnp.zeros((16,), jnp.float32))


@functools.partial(
    pl.kernel,
    out_type=jax.ShapeDtypeStruct((NC, NP), jnp.float32),
    mesh=_mesh,
    scratch_types=[
        pltpu.VMEM((KB, EB), jnp.int32),
        pltpu.VMEM((EB, H), jnp.float32),
        pltpu.VMEM((EB, H), jnp.float32),
        pltpu.VMEM((RPT,), jnp.float32),
        pltpu.VMEM_SHARED((NP, H), jnp.float32),
        pltpu.SemaphoreType.DMA,
        pltpu.SemaphoreType.DMA,
    ],
)
def _sc_degrees(edges_hbm, ones_hbm, zeros_hbm, nrm_out,
                idxbuf, ones, tbuf, nbuf, acc, sems0, sems1):
    c = lax.axis_index("c")
    s = lax.axis_index("s")
    pltpu.sync_copy(ones_hbm, ones)
    pltpu.sync_copy(zeros_hbm.at[pl.ds(s * RPT, RPT)], acc.at[pl.ds(s * RPT, RPT)])
    plsc.subcore_barrier()

    for h in range(2):
        pltpu.sync_copy(edges_hbm.at[c].at[pl.ds(s * 2 * KB + h * KB, KB)],
                        idxbuf)

        def body(i, carry):
            for b in range(2):
                j = 2 * i + b
                sem = sems0 if b == 0 else sems1

                @pl.when(j >= 2)
                def _():
                    pltpu.make_async_copy(ones, acc.at[idxbuf.at[0]], sem).wait()

                pltpu.async_copy(ones, acc.at[idxbuf.at[j]], sem, add=True)
            return carry

        lax.fori_loop(0, KB // 2, body, 0)
        pltpu.make_async_copy(ones, acc.at[idxbuf.at[0]], sems0).wait()
        pltpu.make_async_copy(ones, acc.at[idxbuf.at[0]], sems1).wait()

    plsc.subcore_barrier()

    iota16 = lax.iota(jnp.int32, 16)

    def extract(k, carry):
        pltpu.sync_copy(acc.at[pl.ds(s * RPT + k * EB, EB)], tbuf)
        for g in range(EB // 16):
            cvec = jnp.zeros((16,), jnp.float32)
            for r in range(16):
                v = tbuf[g * 16 + r, pl.ds(0, 16)]
                cvec = jnp.where(iota16 == r, v, cvec)
            nbuf[pl.ds(k * EB + g * 16, 16)] = _rsqrt16(cvec)
        return carry

    lax.fori_loop(0, RPT // EB, extract, 0)
    pltpu.sync_copy(nbuf, nrm_out.at[c].at[pl.ds(s * RPT, RPT)])


@functools.partial(
    pl.kernel,
    out_type=jax.ShapeDtypeStruct((NC, NP, H), jnp.float32),
    mesh=_mesh,
    scratch_types=[
        pltpu.VMEM((KB, EB), jnp.int32),
        pltpu.VMEM((4, EB), jnp.int32),
        pltpu.VMEM((EB, H), jnp.float32),
        pltpu.VMEM((EB, H), jnp.float32),
        pltpu.VMEM_SHARED((NP, H), jnp.float32),
        pltpu.SemaphoreType.DMA,
        pltpu.SemaphoreType.DMA,
        pltpu.SemaphoreType.DMA,
        pltpu.SemaphoreType.DMA,
    ],
)
def _sc_agg(t_hbm, src_hbm, dst_hbm, zeros_hbm, out_hbm,
            sidx, didx, rows0, rows1, acc, semr, semi, sems0, sems1):
    c = lax.axis_index("c")
    s = lax.axis_index("s")
    w = c * NS + s
    pltpu.sync_copy(zeros_hbm.at[pl.ds(s * RPT, RPT)], acc.at[pl.ds(s * RPT, RPT)])
    pltpu.sync_copy(src_hbm.at[pl.ds(w * KB, KB)], sidx)
    plsc.subcore_barrier()

    pltpu.async_copy(t_hbm.at[sidx.at[0]], rows0, semr)
    pltpu.async_copy(dst_hbm.at[w * KB], didx.at[pl.ds(0, 1)], semi)

    def body(i, carry):
        for b in range(2):
            j = 2 * i + b
            cur, nxt = (rows0, rows1) if b == 0 else (rows1, rows0)
            scur, snxt = (sems0, sems1) if b == 0 else (sems1, sems0)
            jm = lax.rem(j, 4)
            pltpu.make_async_copy(t_hbm.at[sidx.at[j]], cur, semr).wait()
            pltpu.make_async_copy(dst_hbm.at[w * KB + j],
                                  didx.at[pl.ds(jm, 1)], semi).wait()
            pltpu.async_copy(cur, acc.at[didx.at[jm]], scur, add=True)

            @pl.when(j >= 1)
            def _():
                pltpu.make_async_copy(nxt, acc.at[didx.at[0]], snxt).wait()

            @pl.when(j + 1 < KB)
            def _():
                jn = lax.rem(j + 1, 4)
                pltpu.async_copy(t_hbm.at[sidx.at[j + 1]], nxt, semr)
                pltpu.async_copy(dst_hbm.at[w * KB + j + 1],
                                 didx.at[pl.ds(jn, 1)], semi)

        return carry

    lax.fori_loop(0, KB // 2, body, 0)
    pltpu.make_async_copy(rows1, acc.at[didx.at[0]], sems1).wait()
    plsc.subcore_barrier()
    pltpu.sync_copy(acc.at[pl.ds(s * RPT, RPT)],
                    out_hbm.at[c].at[pl.ds(s * RPT, RPT)])


R = 1024


def _tc1_body(nrm_ref, x_ref, w1_ref, t1_ref):
    ns_col = nrm_ref[0, :][:, None]
    t1_ref[...] = (
        jnp.dot(x_ref[...], w1_ref[...], preferred_element_type=jnp.float32) * ns_col
    )


_tc1 = pl.pallas_call(
    _tc1_body,
    grid=(NP // R,),
    in_specs=[
        pl.BlockSpec((2, R), lambda i: (0, i)),
        pl.BlockSpec((R, D), lambda i: (i, 0)),
        pl.BlockSpec((D, H), lambda i: (0, 0)),
    ],
    out_specs=pl.BlockSpec((R, H), lambda i: (i, 0)),
    out_shape=jax.ShapeDtypeStruct((NP, H), jnp.float32),
)


def _tc_mid_body(aggp_ref, nrm_ref, b_ref, w_ref, t_ref):
    nd_col = nrm_ref[1, :][:, None]
    h = jnp.maximum((aggp_ref[0] + aggp_ref[1]) * nd_col + b_ref[...], 0.0)
    ns_col = nrm_ref[0, :][:, None]
    t_ref[...] = (
        jnp.dot(h, w_ref[...], preferred_element_type=jnp.float32) * ns_col
    )


_tc_mid = pl.pallas_call(
    _tc_mid_body,
    grid=(NP // R,),
    in_specs=[
        pl.BlockSpec((NC, R, H), lambda i: (0, i, 0)),
        pl.BlockSpec((2, R), lambda i: (0, i)),
        pl.BlockSpec((1, H), lambda i: (0, 0)),
        pl.BlockSpec((H, H), lambda i: (0, 0)),
    ],
    out_specs=pl.BlockSpec((R, H), lambda i: (i, 0)),
    out_shape=jax.ShapeDtypeStruct((NP, H), jnp.float32),
)


def _tc3_body(aggp_ref, nrm_ref, b_ref, w3_ref, t3_ref):
    nd_col = nrm_ref[1, :][:, None]
    h = jnp.maximum((aggp_ref[0] + aggp_ref[1]) * nd_col + b_ref[...], 0.0)
    t3 = jnp.sum(h * w3_ref[...], axis=1) * nrm_ref[0, :]
    t3_ref[...] = jnp.broadcast_to(t3[:, None], (R, H))


_tc3 = pl.pallas_call(
    _tc3_body,
    grid=(NP // R,),
    in_specs=[
        pl.BlockSpec((NC, R, H), lambda i: (0, i, 0)),
        pl.BlockSpec((2, R), lambda i: (0, i)),
        pl.BlockSpec((1, H), lambda i: (0, 0)),
        pl.BlockSpec((1, H), lambda i: (0, 0)),
    ],
    out_specs=pl.BlockSpec((R, H), lambda i: (i, 0)),
    out_shape=jax.ShapeDtypeStruct((NP, H), jnp.float32),
)


def _tc4_body(aggs_ref, nrm_ref, b3_ref, y_ref):
    a = aggs_ref[0, :, 0] + aggs_ref[1, :, 0]
    v = a * nrm_ref[1, :] + b3_ref[0, 0]
    y_ref[...] = jnp.maximum(v, 0.0)[:, None]


_tc4 = pl.pallas_call(
    _tc4_body,
    grid=(NP // R,),
    in_specs=[
        pl.BlockSpec((NC, R, H), lambda i: (0, i, 0)),
        pl.BlockSpec((2, R), lambda i: (0, i)),
        pl.BlockSpec((1, 1), lambda i: (0, 0)),
    ],
    out_specs=pl.BlockSpec((R, 1), lambda i: (i, 0)),
    out_shape=jax.ShapeDtypeStruct((NP, 1), jnp.float32),
)


def kernel(features, edge_index, W1, b1, W2, b2, W3, b3):
    x = jnp.zeros((NP, D), jnp.float32).at[:N].set(features)
    padv = PAD + jnp.arange(EP - E, dtype=jnp.int32) % (NP - N)
    srcp = jnp.concatenate([edge_index[0], padv]).reshape(EP // EB, EB)
    dstp = jnp.concatenate([edge_index[1], padv]).reshape(EP // EB, EB)
    edges2 = jnp.stack([srcp, dstp])
    dst3 = dstp.reshape(EP // EB, 1, EB)
    onesH = jnp.ones((EB, H), jnp.float32)
    zH = jnp.zeros((NP, H), jnp.float32)

    nrm = _sc_degrees(edges2, onesH, zH)
    t1 = _tc1(nrm, x, W1)
    agg1 = _sc_agg(t1, srcp, dst3, zH)
    t2 = _tc_mid(agg1, nrm, b1.reshape(1, H), W2)
    agg2 = _sc_agg(t2, srcp, dst3, zH)
    t3 = _tc3(agg2, nrm, b2.reshape(1, H), W3.reshape(1, H))
    agg3 = _sc_agg(t3, srcp, dst3, zH)
    y = _tc4(agg3, nrm, b3.reshape(1, 1))
    return y[:N]


# speedup vs baseline: 1.0509x; 1.0202x over previous
import functools

import jax
import jax.numpy as jnp
from jax import lax
from jax.experimental import pallas as pl
from jax.experimental.pallas import tpu as pltpu
from jax.experimental.pallas import tpu_sc as plsc

N = 10000
E = 320000
D = 128
H = 128

NP = 10240
PAD = N
NC = 2
NS = 16
NW = NC * NS
EB = 128
KB = 80
EP = NW * KB * EB
RPT = NP // NS

_mesh = plsc.VectorSubcoreMesh(
    core_axis_name="c", subcore_axis_name="s", num_cores=NC, num_subcores=NS
)


def _zero_acc(buf, acc, s):
    def zrow(r, carry):
        for u in range(H // 16):
            buf[r, pl.ds(u * 16, 16)] = jnp.zeros((16,), jnp.float32)
        return carry

    lax.fori_loop(0, EB, zrow, 0)
    for q in range(RPT // EB):
        pltpu.sync_copy(buf, acc.at[pl.ds(s * RPT + q * EB, EB)])


@functools.partial(
    pl.kernel,
    out_type=jax.ShapeDtypeStruct((NC, NP, H), jnp.float32),
    mesh=_mesh,
    scratch_types=[
        pltpu.VMEM((KB, EB), jnp.int32),
        pltpu.VMEM((EB, H), jnp.float32),
        pltpu.VMEM((EB, H), jnp.float32),
        pltpu.VMEM_SHARED((NP, H), jnp.float32),
        pltpu.SemaphoreType.DMA,
        pltpu.SemaphoreType.DMA,
    ],
)
def _sc_degrees(edges_hbm, ones_hbm, deg_out,
                idxbuf, ones, zbuf, acc, sems0, sems1):
    c = lax.axis_index("c")
    s = lax.axis_index("s")
    pltpu.sync_copy(ones_hbm, ones)
    _zero_acc(zbuf, acc, s)
    plsc.subcore_barrier()

    for h in range(2):
        pltpu.sync_copy(edges_hbm.at[c].at[pl.ds(s * 2 * KB + h * KB, KB)],
                        idxbuf)

        def body(i, carry):
            for b in range(2):
                j = 2 * i + b
                sem = sems0 if b == 0 else sems1

                @pl.when(j >= 2)
                def _():
                    pltpu.make_async_copy(ones, acc.at[idxbuf.at[0]], sem).wait()

                pltpu.async_copy(ones, acc.at[idxbuf.at[j]], sem, add=True)
            return carry

        lax.fori_loop(0, KB // 2, body, 0)
        pltpu.make_async_copy(ones, acc.at[idxbuf.at[0]], sems0).wait()
        pltpu.make_async_copy(ones, acc.at[idxbuf.at[0]], sems1).wait()

    plsc.subcore_barrier()
    pltpu.sync_copy(acc.at[pl.ds(s * RPT, RPT)],
                    deg_out.at[c].at[pl.ds(s * RPT, RPT)])


@functools.partial(
    pl.kernel,
    out_type=jax.ShapeDtypeStruct((NC, NP, H), jnp.float32),
    mesh=_mesh,
    scratch_types=[
        pltpu.VMEM((KB, EB), jnp.int32),
        pltpu.VMEM((4, EB), jnp.int32),
        pltpu.VMEM((EB, H), jnp.float32),
        pltpu.VMEM((EB, H), jnp.float32),
        pltpu.VMEM_SHARED((NP, H), jnp.float32),
        pltpu.SemaphoreType.DMA,
        pltpu.SemaphoreType.DMA,
        pltpu.SemaphoreType.DMA,
        pltpu.SemaphoreType.DMA,
    ],
)
def _sc_agg(t_hbm, src_hbm, dst_hbm, out_hbm,
            sidx, didx, rows0, rows1, acc, semr, semi, sems0, sems1):
    c = lax.axis_index("c")
    s = lax.axis_index("s")
    w = c * NS + s
    _zero_acc(rows0, acc, s)
    pltpu.sync_copy(src_hbm.at[pl.ds(w * KB, KB)], sidx)
    plsc.subcore_barrier()

    pltpu.async_copy(t_hbm.at[sidx.at[0]], rows0, semr)
    pltpu.async_copy(dst_hbm.at[w * KB], didx.at[pl.ds(0, 1)], semi)

    def body(i, carry):
        for b in range(2):
            j = 2 * i + b
            cur, nxt = (rows0, rows1) if b == 0 else (rows1, rows0)
            scur, snxt = (sems0, sems1) if b == 0 else (sems1, sems0)
            jm = lax.rem(j, 4)
            pltpu.make_async_copy(t_hbm.at[sidx.at[j]], cur, semr).wait()
            pltpu.make_async_copy(dst_hbm.at[w * KB + j],
                                  didx.at[pl.ds(jm, 1)], semi).wait()
            pltpu.async_copy(cur, acc.at[didx.at[jm]], scur, add=True)

            @pl.when(j >= 1)
            def _():
                pltpu.make_async_copy(nxt, acc.at[didx.at[0]], snxt).wait()

            @pl.when(j + 1 < KB)
            def _():
                jn = lax.rem(j + 1, 4)
                pltpu.async_copy(t_hbm.at[sidx.at[j + 1]], nxt, semr)
                pltpu.async_copy(dst_hbm.at[w * KB + j + 1],
                                 didx.at[pl.ds(jn, 1)], semi)

        return carry

    lax.fori_loop(0, KB // 2, body, 0)
    pltpu.make_async_copy(rows1, acc.at[didx.at[0]], sems1).wait()
    plsc.subcore_barrier()
    pltpu.sync_copy(acc.at[pl.ds(s * RPT, RPT)],
                    out_hbm.at[c].at[pl.ds(s * RPT, RPT)])


R = 1024


def _tc1_body(deg_ref, x_ref, w1_ref, nrm_ref, t1_ref):
    dego = deg_ref[0, :, 0]
    degi = deg_ref[1, :, 0]
    deg = jnp.stack([dego, degi])
    nrm = jnp.where(deg > 0, lax.rsqrt(jnp.maximum(deg, 1e-12)), 0.0)
    nrm_ref[...] = nrm
    ns_col = nrm[0, :][:, None]
    t1_ref[...] = (
        jnp.dot(x_ref[...], w1_ref[...], preferred_element_type=jnp.float32) * ns_col
    )


_tc1 = pl.pallas_call(
    _tc1_body,
    grid=(NP // R,),
    in_specs=[
        pl.BlockSpec((NC, R, H), lambda i: (0, i, 0)),
        pl.BlockSpec((R, D), lambda i: (i, 0)),
        pl.BlockSpec((D, H), lambda i: (0, 0)),
    ],
    out_specs=[
        pl.BlockSpec((2, R), lambda i: (0, i)),
        pl.BlockSpec((R, H), lambda i: (i, 0)),
    ],
    out_shape=[
        jax.ShapeDtypeStruct((2, NP), jnp.float32),
        jax.ShapeDtypeStruct((NP, H), jnp.float32),
    ],
)


def _tc_mid_body(aggp_ref, nrm_ref, b_ref, w_ref, t_ref):
    nd_col = nrm_ref[1, :][:, None]
    h = jnp.maximum((aggp_ref[0] + aggp_ref[1]) * nd_col + b_ref[...], 0.0)
    ns_col = nrm_ref[0, :][:, None]
    t_ref[...] = (
        jnp.dot(h, w_ref[...], preferred_element_type=jnp.float32) * ns_col
    )


_tc_mid = pl.pallas_call(
    _tc_mid_body,
    grid=(NP // R,),
    in_specs=[
        pl.BlockSpec((NC, R, H), lambda i: (0, i, 0)),
        pl.BlockSpec((2, R), lambda i: (0, i)),
        pl.BlockSpec((1, H), lambda i: (0, 0)),
        pl.BlockSpec((H, H), lambda i: (0, 0)),
    ],
    out_specs=pl.BlockSpec((R, H), lambda i: (i, 0)),
    out_shape=jax.ShapeDtypeStruct((NP, H), jnp.float32),
)


def _tc3_body(aggp_ref, nrm_ref, b_ref, w3_ref, t3_ref):
    nd_col = nrm_ref[1, :][:, None]
    h = jnp.maximum((aggp_ref[0] + aggp_ref[1]) * nd_col + b_ref[...], 0.0)
    t3 = jnp.sum(h * w3_ref[...], axis=1) * nrm_ref[0, :]
    t3_ref[...] = jnp.broadcast_to(t3[:, None], (R, H))


_tc3 = pl.pallas_call(
    _tc3_body,
    grid=(NP // R,),
    in_specs=[
        pl.BlockSpec((NC, R, H), lambda i: (0, i, 0)),
        pl.BlockSpec((2, R), lambda i: (0, i)),
        pl.BlockSpec((1, H), lambda i: (0, 0)),
        pl.BlockSpec((1, H), lambda i: (0, 0)),
    ],
    out_specs=pl.BlockSpec((R, H), lambda i: (i, 0)),
    out_shape=jax.ShapeDtypeStruct((NP, H), jnp.float32),
)


def _tc4_body(aggs_ref, nrm_ref, b3_ref, y_ref):
    a = aggs_ref[0, :, 0] + aggs_ref[1, :, 0]
    v = a * nrm_ref[1, :] + b3_ref[0, 0]
    y_ref[...] = jnp.maximum(v, 0.0)[:, None]


_tc4 = pl.pallas_call(
    _tc4_body,
    grid=(NP // R,),
    in_specs=[
        pl.BlockSpec((NC, R, H), lambda i: (0, i, 0)),
        pl.BlockSpec((2, R), lambda i: (0, i)),
        pl.BlockSpec((1, 1), lambda i: (0, 0)),
    ],
    out_specs=pl.BlockSpec((R, 1), lambda i: (i, 0)),
    out_shape=jax.ShapeDtypeStruct((NP, 1), jnp.float32),
)


def kernel(features, edge_index, W1, b1, W2, b2, W3, b3):
    x = jnp.zeros((NP, D), jnp.float32).at[:N].set(features)
    padv = PAD + jnp.arange(EP - E, dtype=jnp.int32) % (NP - N)
    srcp = jnp.concatenate([edge_index[0], padv]).reshape(EP // EB, EB)
    dstp = jnp.concatenate([edge_index[1], padv]).reshape(EP // EB, EB)
    edges2 = jnp.stack([srcp, dstp])
    dst3 = dstp.reshape(EP // EB, 1, EB)
    onesH = jnp.ones((EB, H), jnp.float32)

    deg = _sc_degrees(edges2, onesH)
    nrm, t1 = _tc1(deg, x, W1)
    agg1 = _sc_agg(t1, srcp, dst3)
    t2 = _tc_mid(agg1, nrm, b1.reshape(1, H), W2)
    agg2 = _sc_agg(t2, srcp, dst3)
    t3 = _tc3(agg2, nrm, b2.reshape(1, H), W3.reshape(1, H))
    agg3 = _sc_agg(t3, srcp, dst3)
    y = _tc4(agg3, nrm, b3.reshape(1, 1))
    return y[:N]
